# Initial kernel scaffold; baseline (speedup 1.0000x reference)
import jax
import jax.numpy as jnp
from jax.experimental import pallas as pl


def kernel(inputs, face, spatial_weights, depth_weights, biases, bn_gamma, bn_beta):
    raise NotImplementedError("write your pallas kernel here")


# speedup vs baseline: 5.1681x; 5.1681x over previous
import functools

import jax
import jax.numpy as jnp
from jax import lax
from jax.experimental import pallas as pl
from jax.experimental.pallas import tpu as pltpu
from jax.experimental.pallas import tpu_sc as plsc

F_ = 320000
C_ = 128
NC_ = 2
NS_ = 16
NW_ = NC_ * NS_
CHUNK_ = 128
NFULL_ = 78
NEXTRA_ = (F_ - NW_ * NFULL_ * CHUNK_) // CHUNK_

BT_ = 2000
NB_ = F_ // BT_


def _sc_gather_body(inp_hbm, i0_hbm, i1_hbm, i2_hbm,
                    g0_hbm, g1_hbm, g2_hbm,
                    iv0, iv1, iv2, rv0, rv1, rv2, s0, s1, s2):
  wid = lax.axis_index("s") * NC_ + lax.axis_index("c")

  def chunk(base):
    pltpu.sync_copy(i0_hbm.at[pl.ds(base, CHUNK_)], iv0)
    pltpu.sync_copy(i1_hbm.at[pl.ds(base, CHUNK_)], iv1)
    pltpu.sync_copy(i2_hbm.at[pl.ds(base, CHUNK_)], iv2)
    c0 = pltpu.async_copy(inp_hbm.at[iv0], rv0, s0)
    c1 = pltpu.async_copy(inp_hbm.at[iv1], rv1, s1)
    c2 = pltpu.async_copy(inp_hbm.at[iv2], rv2, s2)
    c0.wait()
    c1.wait()
    c2.wait()
    pltpu.sync_copy(rv0, g0_hbm.at[pl.ds(base, CHUNK_)])
    pltpu.sync_copy(rv1, g1_hbm.at[pl.ds(base, CHUNK_)])
    pltpu.sync_copy(rv2, g2_hbm.at[pl.ds(base, CHUNK_)])

  def body(j, carry):
    chunk((wid * NFULL_ + j) * CHUNK_)
    return carry

  lax.fori_loop(0, NFULL_, body, 0)

  @pl.when(wid < NEXTRA_)
  def _():
    chunk((NW_ * NFULL_ + wid) * CHUNK_)


@functools.lru_cache(maxsize=None)
def _get_sc_gather():
  return pl.kernel(
    out_type=(
        jax.ShapeDtypeStruct((F_, C_), jnp.float32),
        jax.ShapeDtypeStruct((F_, C_), jnp.float32),
        jax.ShapeDtypeStruct((F_, C_), jnp.float32),
    ),
    mesh=plsc.VectorSubcoreMesh(core_axis_name="c", subcore_axis_name="s"),
    scratch_types=[
        pltpu.VMEM((CHUNK_,), jnp.int32),
        pltpu.VMEM((CHUNK_,), jnp.int32),
        pltpu.VMEM((CHUNK_,), jnp.int32),
        pltpu.VMEM((CHUNK_, C_), jnp.float32),
        pltpu.VMEM((CHUNK_, C_), jnp.float32),
        pltpu.VMEM((CHUNK_, C_), jnp.float32),
        pltpu.SemaphoreType.DMA,
        pltpu.SemaphoreType.DMA,
        pltpu.SemaphoreType.DMA,
    ],
  )(_sc_gather_body)


def _a_body(g0, g1, g2, sw, dw, bb, pre, stats):
  w = dw[...]
  w0 = sw[0, :][:, None] * w
  w1 = sw[1, :][:, None] * w
  w2 = sw[2, :][:, None] * w
  acc = jnp.dot(g0[...], w0, preferred_element_type=jnp.float32)
  acc = acc + jnp.dot(g1[...], w1, preferred_element_type=jnp.float32)
  acc = acc + jnp.dot(g2[...], w2, preferred_element_type=jnp.float32)
  acc = acc + bb[0, :][None, :]
  r = jnp.maximum(acc, 0.0)
  pre[...] = r
  s = jnp.sum(r, axis=0)
  s2 = jnp.sum(r * r, axis=0)
  upd = jnp.concatenate(
      [s[None, :], s2[None, :], jnp.zeros((6, C_), jnp.float32)], axis=0)

  @pl.when(pl.program_id(0) == 0)
  def _():
    stats[...] = upd

  @pl.when(pl.program_id(0) != 0)
  def _():
    stats[...] = stats[...] + upd


def _b_body(pre, stats, gb, out):
  mean = stats[0, :] / F_
  var = stats[1, :] / F_ - mean * mean
  inv = gb[0, :] / jnp.sqrt(var + 1e-5)
  out[...] = (pre[...] - mean[None, :]) * inv[None, :] + gb[1, :][None, :]


def kernel(inputs, face, spatial_weights, depth_weights, biases,
           bn_gamma, bn_beta):
  face32 = face.astype(jnp.int32)
  ft = face32.T
  i0 = ft[0]
  i1 = ft[1]
  i2 = ft[2]

  g0, g1, g2 = _get_sc_gather()(inputs, i0, i1, i2)

  sw8 = jnp.pad(spatial_weights[:, :, 0], ((0, 5), (0, 0)))
  bb8 = jnp.pad(biases, ((0, 7), (0, 0)))

  pre, stats = pl.pallas_call(
      _a_body,
      grid=(NB_,),
      in_specs=[
          pl.BlockSpec((BT_, C_), lambda i: (i, 0)),
          pl.BlockSpec((BT_, C_), lambda i: (i, 0)),
          pl.BlockSpec((BT_, C_), lambda i: (i, 0)),
          pl.BlockSpec((8, C_), lambda i: (0, 0)),
          pl.BlockSpec((C_, C_), lambda i: (0, 0)),
          pl.BlockSpec((8, C_), lambda i: (0, 0)),
      ],
      out_specs=[
          pl.BlockSpec((BT_, C_), lambda i: (i, 0)),
          pl.BlockSpec((8, C_), lambda i: (0, 0)),
      ],
      out_shape=[
          jax.ShapeDtypeStruct((F_, C_), jnp.float32),
          jax.ShapeDtypeStruct((8, C_), jnp.float32),
      ],
  )(g0, g1, g2, sw8, depth_weights, bb8)

  gb8 = jnp.pad(jnp.stack([bn_gamma, bn_beta]), ((0, 6), (0, 0)))

  out = pl.pallas_call(
      _b_body,
      grid=(NB_,),
      in_specs=[
          pl.BlockSpec((BT_, C_), lambda i: (i, 0)),
          pl.BlockSpec((8, C_), lambda i: (0, 0)),
          pl.BlockSpec((8, C_), lambda i: (0, 0)),
      ],
      out_specs=pl.BlockSpec((BT_, C_), lambda i: (i, 0)),
      out_shape=jax.ShapeDtypeStruct((F_, C_), jnp.float32),
  )(pre, stats, gb8)

  return out


# speedup vs baseline: 6.2342x; 1.2063x over previous
import functools

import jax
import jax.numpy as jnp
from jax import lax
from jax.experimental import pallas as pl
from jax.experimental.pallas import tpu as pltpu
from jax.experimental.pallas import tpu_sc as plsc

F_ = 320000
C_ = 128
NC_ = 2
NS_ = 16
NW_ = NC_ * NS_
CHUNK_ = 128
NFULL_ = 78
NEXTRA_ = (F_ - NW_ * NFULL_ * CHUNK_) // CHUNK_

BT_ = 2000
NB_ = F_ // BT_


ROWS_W_ = NFULL_ * CHUNK_


def _sc_gather_body(inp_hbm, i0_hbm, i1_hbm, i2_hbm,
                    g0_hbm, g1_hbm, g2_hbm,
                    iv0, iv1, iv2,
                    ra0, ra1, ra2, rb0, rb1, rb2, sa, sb):
  wid = lax.axis_index("s") * NC_ + lax.axis_index("c")
  wbase = wid * ROWS_W_

  pltpu.sync_copy(i0_hbm.at[pl.ds(wbase, ROWS_W_)], iv0)
  pltpu.sync_copy(i1_hbm.at[pl.ds(wbase, ROWS_W_)], iv1)
  pltpu.sync_copy(i2_hbm.at[pl.ds(wbase, ROWS_W_)], iv2)

  def issue(bufs, sem, j):
    off = j * CHUNK_
    pltpu.async_copy(inp_hbm.at[iv0.at[pl.ds(off, CHUNK_)]], bufs[0], sem)
    pltpu.async_copy(inp_hbm.at[iv1.at[pl.ds(off, CHUNK_)]], bufs[1], sem)
    pltpu.async_copy(inp_hbm.at[iv2.at[pl.ds(off, CHUNK_)]], bufs[2], sem)

  def drain(bufs, sem, j):
    off = j * CHUNK_
    pltpu.make_async_copy(inp_hbm.at[iv0.at[pl.ds(off, CHUNK_)]], bufs[0],
                          sem).wait()
    pltpu.make_async_copy(inp_hbm.at[iv1.at[pl.ds(off, CHUNK_)]], bufs[1],
                          sem).wait()
    pltpu.make_async_copy(inp_hbm.at[iv2.at[pl.ds(off, CHUNK_)]], bufs[2],
                          sem).wait()

  def store(bufs, j):
    base = wbase + j * CHUNK_
    pltpu.sync_copy(bufs[0], g0_hbm.at[pl.ds(base, CHUNK_)])
    pltpu.sync_copy(bufs[1], g1_hbm.at[pl.ds(base, CHUNK_)])
    pltpu.sync_copy(bufs[2], g2_hbm.at[pl.ds(base, CHUNK_)])

  bufs_a = (ra0, ra1, ra2)
  bufs_b = (rb0, rb1, rb2)

  issue(bufs_a, sa, 0)

  def body(i, carry):
    j0 = 2 * i
    issue(bufs_b, sb, j0 + 1)
    drain(bufs_a, sa, j0)
    store(bufs_a, j0)

    @pl.when(j0 + 2 < NFULL_)
    def _():
      issue(bufs_a, sa, j0 + 2)

    drain(bufs_b, sb, j0 + 1)
    store(bufs_b, j0 + 1)
    return carry

  lax.fori_loop(0, NFULL_ // 2, body, 0)

  @pl.when(wid < NEXTRA_)
  def _():
    base = (NW_ * NFULL_ + wid) * CHUNK_
    pltpu.sync_copy(i0_hbm.at[pl.ds(base, CHUNK_)], iv0.at[pl.ds(0, CHUNK_)])
    pltpu.sync_copy(i1_hbm.at[pl.ds(base, CHUNK_)], iv1.at[pl.ds(0, CHUNK_)])
    pltpu.sync_copy(i2_hbm.at[pl.ds(base, CHUNK_)], iv2.at[pl.ds(0, CHUNK_)])
    issue(bufs_a, sa, 0)
    drain(bufs_a, sa, 0)
    pltpu.sync_copy(ra0, g0_hbm.at[pl.ds(base, CHUNK_)])
    pltpu.sync_copy(ra1, g1_hbm.at[pl.ds(base, CHUNK_)])
    pltpu.sync_copy(ra2, g2_hbm.at[pl.ds(base, CHUNK_)])


@functools.lru_cache(maxsize=None)
def _get_sc_gather():
  return pl.kernel(
    out_type=(
        jax.ShapeDtypeStruct((F_, C_), jnp.float32),
        jax.ShapeDtypeStruct((F_, C_), jnp.float32),
        jax.ShapeDtypeStruct((F_, C_), jnp.float32),
    ),
    mesh=plsc.VectorSubcoreMesh(core_axis_name="c", subcore_axis_name="s"),
    scratch_types=[
        pltpu.VMEM((ROWS_W_,), jnp.int32),
        pltpu.VMEM((ROWS_W_,), jnp.int32),
        pltpu.VMEM((ROWS_W_,), jnp.int32),
        pltpu.VMEM((CHUNK_, C_), jnp.float32),
        pltpu.VMEM((CHUNK_, C_), jnp.float32),
        pltpu.VMEM((CHUNK_, C_), jnp.float32),
        pltpu.VMEM((CHUNK_, C_), jnp.float32),
        pltpu.VMEM((CHUNK_, C_), jnp.float32),
        pltpu.VMEM((CHUNK_, C_), jnp.float32),
        pltpu.SemaphoreType.DMA,
        pltpu.SemaphoreType.DMA,
    ],
  )(_sc_gather_body)


def _a_body(g0, g1, g2, sw, dw, bb, pre, stats):
  w = dw[...]
  w0 = sw[0, :][:, None] * w
  w1 = sw[1, :][:, None] * w
  w2 = sw[2, :][:, None] * w
  acc = jnp.dot(g0[...], w0, preferred_element_type=jnp.float32)
  acc = acc + jnp.dot(g1[...], w1, preferred_element_type=jnp.float32)
  acc = acc + jnp.dot(g2[...], w2, preferred_element_type=jnp.float32)
  acc = acc + bb[0, :][None, :]
  r = jnp.maximum(acc, 0.0)
  pre[...] = r
  s = jnp.sum(r, axis=0)
  s2 = jnp.sum(r * r, axis=0)
  upd = jnp.concatenate(
      [s[None, :], s2[None, :], jnp.zeros((6, C_), jnp.float32)], axis=0)

  @pl.when(pl.program_id(0) == 0)
  def _():
    stats[...] = upd

  @pl.when(pl.program_id(0) != 0)
  def _():
    stats[...] = stats[...] + upd


def _b_body(pre, stats, gb, out):
  mean = stats[0, :] / F_
  var = stats[1, :] / F_ - mean * mean
  inv = gb[0, :] / jnp.sqrt(var + 1e-5)
  out[...] = (pre[...] - mean[None, :]) * inv[None, :] + gb[1, :][None, :]


def kernel(inputs, face, spatial_weights, depth_weights, biases,
           bn_gamma, bn_beta):
  face32 = face.astype(jnp.int32)
  ft = face32.T
  i0 = ft[0]
  i1 = ft[1]
  i2 = ft[2]

  g0, g1, g2 = _get_sc_gather()(inputs, i0, i1, i2)

  sw8 = jnp.pad(spatial_weights[:, :, 0], ((0, 5), (0, 0)))
  bb8 = jnp.pad(biases, ((0, 7), (0, 0)))

  pre, stats = pl.pallas_call(
      _a_body,
      grid=(NB_,),
      in_specs=[
          pl.BlockSpec((BT_, C_), lambda i: (i, 0)),
          pl.BlockSpec((BT_, C_), lambda i: (i, 0)),
          pl.BlockSpec((BT_, C_), lambda i: (i, 0)),
          pl.BlockSpec((8, C_), lambda i: (0, 0)),
          pl.BlockSpec((C_, C_), lambda i: (0, 0)),
          pl.BlockSpec((8, C_), lambda i: (0, 0)),
      ],
      out_specs=[
          pl.BlockSpec((BT_, C_), lambda i: (i, 0)),
          pl.BlockSpec((8, C_), lambda i: (0, 0)),
      ],
      out_shape=[
          jax.ShapeDtypeStruct((F_, C_), jnp.float32),
          jax.ShapeDtypeStruct((8, C_), jnp.float32),
      ],
  )(g0, g1, g2, sw8, depth_weights, bb8)

  gb8 = jnp.pad(jnp.stack([bn_gamma, bn_beta]), ((0, 6), (0, 0)))

  out = pl.pallas_call(
      _b_body,
      grid=(NB_,),
      in_specs=[
          pl.BlockSpec((BT_, C_), lambda i: (i, 0)),
          pl.BlockSpec((8, C_), lambda i: (0, 0)),
          pl.BlockSpec((8, C_), lambda i: (0, 0)),
      ],
      out_specs=pl.BlockSpec((BT_, C_), lambda i: (i, 0)),
      out_shape=jax.ShapeDtypeStruct((F_, C_), jnp.float32),
  )(pre, stats, gb8)

  return out


# speedup vs baseline: 8.4062x; 1.3484x over previous
import functools

import jax
import jax.numpy as jnp
from jax import lax
from jax.experimental import pallas as pl
from jax.experimental.pallas import tpu as pltpu
from jax.experimental.pallas import tpu_sc as plsc

N_ = 10000
F_ = 320000
C_ = 128
NC_ = 2
NS_ = 16
NW_ = NC_ * NS_
CHUNK_ = 128
NFULL_ = 78
NEXTRA_ = (F_ - NW_ * NFULL_ * CHUNK_) // CHUNK_
ROWS_W_ = NFULL_ * CHUNK_

BT_ = 2000
NB_ = F_ // BT_


def _c_body(inp, sw, t):
  x = inp[...]
  t[pl.ds(0, N_), :] = x * sw[0, :][None, :]
  t[pl.ds(N_, N_), :] = x * sw[1, :][None, :]
  t[pl.ds(2 * N_, N_), :] = x * sw[2, :][None, :]


def _sc_body(t_hbm, i0_hbm, i1_hbm, i2_hbm, v2f_hbm,
             iv0, iv1, iv2,
             ra0, ra1, ra2, rb0, rb1, rb2, sa, sb):
  wid = lax.axis_index("s") * NC_ + lax.axis_index("c")
  wbase = wid * ROWS_W_

  pltpu.sync_copy(i0_hbm.at[pl.ds(wbase, ROWS_W_)], iv0)
  pltpu.sync_copy(i1_hbm.at[pl.ds(wbase, ROWS_W_)], iv1)
  pltpu.sync_copy(i2_hbm.at[pl.ds(wbase, ROWS_W_)], iv2)

  def issue(bufs, sem, j):
    off = j * CHUNK_
    pltpu.async_copy(t_hbm.at[iv0.at[pl.ds(off, CHUNK_)]], bufs[0], sem)
    pltpu.async_copy(t_hbm.at[iv1.at[pl.ds(off, CHUNK_)]], bufs[1], sem)
    pltpu.async_copy(t_hbm.at[iv2.at[pl.ds(off, CHUNK_)]], bufs[2], sem)

  def drain(bufs, sem, j):
    off = j * CHUNK_
    pltpu.make_async_copy(t_hbm.at[iv0.at[pl.ds(off, CHUNK_)]], bufs[0],
                          sem).wait()
    pltpu.make_async_copy(t_hbm.at[iv1.at[pl.ds(off, CHUNK_)]], bufs[1],
                          sem).wait()
    pltpu.make_async_copy(t_hbm.at[iv2.at[pl.ds(off, CHUNK_)]], bufs[2],
                          sem).wait()

  def combine(bufs):
    def row(r, carry):
      for s in range(C_ // 16):
        sl = pl.ds(s * 16, 16)
        bufs[0][r, sl] = bufs[0][r, sl] + bufs[1][r, sl] + bufs[2][r, sl]
      return carry

    lax.fori_loop(0, CHUNK_, row, 0)

  def store(bufs, base):
    pltpu.sync_copy(bufs[0], v2f_hbm.at[pl.ds(base, CHUNK_)])

  bufs_a = (ra0, ra1, ra2)
  bufs_b = (rb0, rb1, rb2)

  issue(bufs_a, sa, 0)

  def body(i, carry):
    j0 = 2 * i
    issue(bufs_b, sb, j0 + 1)
    drain(bufs_a, sa, j0)
    combine(bufs_a)
    store(bufs_a, wbase + j0 * CHUNK_)

    @pl.when(j0 + 2 < NFULL_)
    def _():
      issue(bufs_a, sa, j0 + 2)

    drain(bufs_b, sb, j0 + 1)
    combine(bufs_b)
    store(bufs_b, wbase + (j0 + 1) * CHUNK_)
    return carry

  lax.fori_loop(0, NFULL_ // 2, body, 0)

  @pl.when(wid < NEXTRA_)
  def _():
    base = (NW_ * NFULL_ + wid) * CHUNK_
    pltpu.sync_copy(i0_hbm.at[pl.ds(base, CHUNK_)], iv0.at[pl.ds(0, CHUNK_)])
    pltpu.sync_copy(i1_hbm.at[pl.ds(base, CHUNK_)], iv1.at[pl.ds(0, CHUNK_)])
    pltpu.sync_copy(i2_hbm.at[pl.ds(base, CHUNK_)], iv2.at[pl.ds(0, CHUNK_)])
    issue(bufs_a, sa, 0)
    drain(bufs_a, sa, 0)
    combine(bufs_a)
    store(bufs_a, base)


@functools.lru_cache(maxsize=None)
def _get_sc_combine():
  return pl.kernel(
    out_type=jax.ShapeDtypeStruct((F_, C_), jnp.float32),
    mesh=plsc.VectorSubcoreMesh(core_axis_name="c", subcore_axis_name="s"),
    scratch_types=[
        pltpu.VMEM((ROWS_W_,), jnp.int32),
        pltpu.VMEM((ROWS_W_,), jnp.int32),
        pltpu.VMEM((ROWS_W_,), jnp.int32),
        pltpu.VMEM((CHUNK_, C_), jnp.float32),
        pltpu.VMEM((CHUNK_, C_), jnp.float32),
        pltpu.VMEM((CHUNK_, C_), jnp.float32),
        pltpu.VMEM((CHUNK_, C_), jnp.float32),
        pltpu.VMEM((CHUNK_, C_), jnp.float32),
        pltpu.VMEM((CHUNK_, C_), jnp.float32),
        pltpu.SemaphoreType.DMA,
        pltpu.SemaphoreType.DMA,
    ],
  )(_sc_body)


def _a_body(v2f, dw, bb, stats):
  acc = jnp.dot(v2f[...], dw[...], preferred_element_type=jnp.float32)
  acc = acc + bb[0, :][None, :]
  r = jnp.maximum(acc, 0.0)
  s = jnp.sum(r, axis=0)
  s2 = jnp.sum(r * r, axis=0)
  upd = jnp.concatenate(
      [s[None, :], s2[None, :], jnp.zeros((6, C_), jnp.float32)], axis=0)

  @pl.when(pl.program_id(0) == 0)
  def _():
    stats[...] = upd

  @pl.when(pl.program_id(0) != 0)
  def _():
    stats[...] = stats[...] + upd


def _b_body(v2f, dw, bb, stats, gb, out):
  acc = jnp.dot(v2f[...], dw[...], preferred_element_type=jnp.float32)
  acc = acc + bb[0, :][None, :]
  r = jnp.maximum(acc, 0.0)
  mean = stats[0, :] / F_
  var = stats[1, :] / F_ - mean * mean
  inv = gb[0, :] / jnp.sqrt(var + 1e-5)
  out[...] = (r - mean[None, :]) * inv[None, :] + gb[1, :][None, :]


def kernel(inputs, face, spatial_weights, depth_weights, biases,
           bn_gamma, bn_beta):
  face32 = face.astype(jnp.int32)
  ft = face32.T
  i0 = ft[0]
  i1 = ft[1] + N_
  i2 = ft[2] + 2 * N_

  sw8 = jnp.pad(spatial_weights[:, :, 0], ((0, 5), (0, 0)))
  bb8 = jnp.pad(biases, ((0, 7), (0, 0)))
  gb8 = jnp.pad(jnp.stack([bn_gamma, bn_beta]), ((0, 6), (0, 0)))

  t = pl.pallas_call(
      _c_body,
      in_specs=[
          pl.BlockSpec((N_, C_), lambda: (0, 0)),
          pl.BlockSpec((8, C_), lambda: (0, 0)),
      ],
      out_specs=pl.BlockSpec((3 * N_, C_), lambda: (0, 0)),
      out_shape=jax.ShapeDtypeStruct((3 * N_, C_), jnp.float32),
  )(inputs, sw8)

  v2f = _get_sc_combine()(t, i0, i1, i2)

  stats = pl.pallas_call(
      _a_body,
      grid=(NB_,),
      in_specs=[
          pl.BlockSpec((BT_, C_), lambda i: (i, 0)),
          pl.BlockSpec((C_, C_), lambda i: (0, 0)),
          pl.BlockSpec((8, C_), lambda i: (0, 0)),
      ],
      out_specs=pl.BlockSpec((8, C_), lambda i: (0, 0)),
      out_shape=jax.ShapeDtypeStruct((8, C_), jnp.float32),
  )(v2f, depth_weights, bb8)

  out = pl.pallas_call(
      _b_body,
      grid=(NB_,),
      in_specs=[
          pl.BlockSpec((BT_, C_), lambda i: (i, 0)),
          pl.BlockSpec((C_, C_), lambda i: (0, 0)),
          pl.BlockSpec((8, C_), lambda i: (0, 0)),
          pl.BlockSpec((8, C_), lambda i: (0, 0)),
          pl.BlockSpec((8, C_), lambda i: (0, 0)),
      ],
      out_specs=pl.BlockSpec((BT_, C_), lambda i: (i, 0)),
      out_shape=jax.ShapeDtypeStruct((F_, C_), jnp.float32),
  )(v2f, depth_weights, bb8, stats, gb8)

  return out
